# Initial kernel scaffold; baseline (speedup 1.0000x reference)
import jax
import jax.numpy as jnp
from jax.experimental import pallas as pl


def kernel(x, edge_index, edge_attr, W, b):
    raise NotImplementedError("write your pallas kernel here")


# speedup vs baseline: 12.2016x; 12.2016x over previous
import functools

import jax
import jax.numpy as jnp
from jax import lax
from jax.experimental import pallas as pl
from jax.experimental.pallas import tpu as pltpu
from jax.experimental.pallas import tpu_sc as plsc

N_NODES = 100000
D_FEAT = 128
D_EDGE = 16
E_TOTAL = 3200000

NC = 2
NS = 16
NW = NC * NS
ROWS_PER_TILE = 6256
N_PAD = NS * ROWS_PER_TILE
E_HALF = E_TOTAL // NC
EDGES_PER_TILE = E_TOTAL // NW
EC = 2000
NJ1 = E_HALF // EC
NJ2 = EDGES_PER_TILE // EC

BLK = 2000


def _sc_segment_sum(col, attr_t, zeros_flat):
    mesh = plsc.VectorSubcoreMesh(core_axis_name="c", subcore_axis_name="s")

    @functools.partial(
        pl.kernel,
        mesh=mesh,
        out_type=[
            jax.ShapeDtypeStruct((NW, N_PAD), jnp.float32),
            jax.ShapeDtypeStruct((NW, N_PAD), jnp.float32),
        ],
        scratch_types=[
            pltpu.VMEM((EC,), jnp.int32),
            pltpu.VMEM((EC,), jnp.int32),
            pltpu.VMEM((EC,), jnp.float32),
            pltpu.VMEM((EC,), jnp.float32),
            pltpu.VMEM((N_PAD,), jnp.float32),
            pltpu.SemaphoreType.DMA,
            pltpu.SemaphoreType.DMA,
            pltpu.SemaphoreType.DMA,
            pltpu.SemaphoreType.DMA,
        ],
        compiler_params=pltpu.CompilerParams(needs_layout_passes=False),
    )
    def k(col_h, attrt_h, z_h, sums_h, cnts_h,
          idxb0, idxb1, valb0, valb1, hist_v, si0, si1, sv0, sv1):
        c = lax.axis_index("c")
        s = lax.axis_index("s")
        w = c * NS + s
        idxbufs = (idxb0, idxb1)
        valbufs = (valb0, valb1)
        sems_i = (si0, si1)
        sems_v = (sv0, sv1)

        pltpu.sync_copy(z_h, hist_v)
        ebase = c * E_HALF
        vbase = s * E_TOTAL + ebase

        def start1(j, b):
            pltpu.async_copy(col_h.at[pl.ds(ebase + j * EC, EC)],
                             idxbufs[b], sems_i[b])
            pltpu.async_copy(attrt_h.at[pl.ds(vbase + j * EC, EC)],
                             valbufs[b], sems_v[b])

        def wait1(j, b):
            pltpu.make_async_copy(col_h.at[pl.ds(ebase + j * EC, EC)],
                                  idxbufs[b], sems_i[b]).wait()
            pltpu.make_async_copy(attrt_h.at[pl.ds(vbase + j * EC, EC)],
                                  valbufs[b], sems_v[b]).wait()

        start1(0, 0)

        def body1(j2, carry):
            for b in (0, 1):
                j = j2 * 2 + b

                @pl.when(j + 1 < NJ1)
                def _():
                    start1(j + 1, 1 - b)

                wait1(j, b)
                for kk in range(EC // 16):
                    iv = idxbufs[b][pl.ds(kk * 16, 16)]
                    vv = valbufs[b][pl.ds(kk * 16, 16)]
                    plsc.addupdate_scatter(hist_v, [iv], vv)
            return carry

        lax.fori_loop(0, NJ1 // 2, body1, 0)
        pltpu.sync_copy(hist_v, sums_h.at[w])

        pltpu.sync_copy(z_h, hist_v)
        sbase = w * EDGES_PER_TILE
        ones16 = jnp.full((16,), 1.0, dtype=jnp.float32)

        def start2(j, b):
            pltpu.async_copy(col_h.at[pl.ds(sbase + j * EC, EC)],
                             idxbufs[b], sems_i[b])

        def wait2(j, b):
            pltpu.make_async_copy(col_h.at[pl.ds(sbase + j * EC, EC)],
                                  idxbufs[b], sems_i[b]).wait()

        start2(0, 0)

        def body2(j2, carry):
            for b in (0, 1):
                j = j2 * 2 + b

                @pl.when(j + 1 < NJ2)
                def _():
                    start2(j + 1, 1 - b)

                wait2(j, b)
                for kk in range(EC // 16):
                    iv = idxbufs[b][pl.ds(kk * 16, 16)]
                    plsc.addupdate_scatter(hist_v, [iv], ones16)
            return carry

        lax.fori_loop(0, NJ2 // 2, body2, 0)
        pltpu.sync_copy(hist_v, cnts_h.at[w])

    return k(col, attr_t, zeros_flat)


def _tc_combine(x, s0, s1, cnts, wx_t, wa_t, b2):
    def body(x_r, s0_r, s1_r, c_r, wx_r, wa_r, b_r, o_r):
        ones_col = jnp.ones((NW, 1), dtype=jnp.float32)
        cnt = jnp.dot(c_r[...], ones_col, preferred_element_type=jnp.float32)
        inv = 1.0 / jnp.maximum(cnt, 1.0)
        agg = (s0_r[...] + s1_r[...]) * inv
        acc = jnp.dot(x_r[...], wx_r[...], preferred_element_type=jnp.float32)
        acc = acc + jnp.dot(agg, wa_r[...], preferred_element_type=jnp.float32)
        o_r[...] = acc + b_r[...]

    grid = N_NODES // BLK
    return pl.pallas_call(
        body,
        grid=(grid,),
        in_specs=[
            pl.BlockSpec((BLK, D_FEAT), lambda i: (i, 0)),
            pl.BlockSpec((BLK, D_EDGE), lambda i: (i, 0)),
            pl.BlockSpec((BLK, D_EDGE), lambda i: (i, 0)),
            pl.BlockSpec((BLK, NW), lambda i: (i, 0)),
            pl.BlockSpec((D_FEAT, D_FEAT), lambda i: (0, 0)),
            pl.BlockSpec((D_EDGE, D_FEAT), lambda i: (0, 0)),
            pl.BlockSpec((1, D_FEAT), lambda i: (0, 0)),
        ],
        out_specs=pl.BlockSpec((BLK, D_FEAT), lambda i: (i, 0)),
        out_shape=jax.ShapeDtypeStruct((N_NODES, D_FEAT), jnp.float32),
    )(x, s0, s1, cnts, wx_t, wa_t, b2)


@jax.jit
def kernel(x, edge_index, edge_attr, W, b):
    col = edge_index[1]
    attr_t = edge_attr.T.reshape(-1)
    zeros_flat = jnp.zeros((N_PAD,), jnp.float32)
    sums_f, cnts_f = _sc_segment_sum(col, attr_t, zeros_flat)
    s0 = sums_f[:NS, :N_NODES].T
    s1 = sums_f[NS:, :N_NODES].T
    cnts = cnts_f[:, :N_NODES].T
    wt = W.T
    return _tc_combine(x, s0, s1, cnts, wt[:D_FEAT], wt[D_FEAT:], b[None, :])


# speedup vs baseline: 14.8939x; 1.2207x over previous
import functools

import jax
import jax.numpy as jnp
from jax import lax
from jax.experimental import pallas as pl
from jax.experimental.pallas import tpu as pltpu
from jax.experimental.pallas import tpu_sc as plsc

N_NODES = 100000
D_FEAT = 128
D_EDGE = 16
E_TOTAL = 3200000

NC = 2
NS = 16
NW = NC * NS
ROWS_PER_TILE = 6256
N_PAD = NS * ROWS_PER_TILE
E_HALF = E_TOTAL // NC
EDGES_PER_TILE = E_TOTAL // NW
EC = 2000
NJ1 = E_HALF // EC
NJ2 = EDGES_PER_TILE // EC

BLK = 2000


def _sc_segment_sum(col, attr_t, zeros_flat):
    mesh = plsc.VectorSubcoreMesh(core_axis_name="c", subcore_axis_name="s")

    @functools.partial(
        pl.kernel,
        mesh=mesh,
        out_type=[
            jax.ShapeDtypeStruct((NW, N_PAD), jnp.float32),
            jax.ShapeDtypeStruct((NW, N_PAD), jnp.float32),
        ],
        scratch_types=[
            pltpu.VMEM((EC,), jnp.int32),
            pltpu.VMEM((EC,), jnp.int32),
            pltpu.VMEM((EC,), jnp.float32),
            pltpu.VMEM((EC,), jnp.float32),
            pltpu.VMEM((N_PAD,), jnp.float32),
            pltpu.SemaphoreType.DMA,
            pltpu.SemaphoreType.DMA,
            pltpu.SemaphoreType.DMA,
            pltpu.SemaphoreType.DMA,
        ],
        compiler_params=pltpu.CompilerParams(needs_layout_passes=False),
    )
    def k(col_h, attrt_h, z_h, sums_h, cnts_h,
          idxb0, idxb1, valb0, valb1, hist_v, si0, si1, sv0, sv1):
        c = lax.axis_index("c")
        s = lax.axis_index("s")
        w = c * NS + s
        idxbufs = (idxb0, idxb1)
        valbufs = (valb0, valb1)
        sems_i = (si0, si1)
        sems_v = (sv0, sv1)

        pltpu.sync_copy(z_h, hist_v)
        ebase = c * E_HALF
        vbase = s * E_TOTAL + ebase

        def start1(j, b):
            pltpu.async_copy(col_h.at[pl.ds(ebase + j * EC, EC)],
                             idxbufs[b], sems_i[b])
            pltpu.async_copy(attrt_h.at[pl.ds(vbase + j * EC, EC)],
                             valbufs[b], sems_v[b])

        def wait1(j, b):
            pltpu.make_async_copy(col_h.at[pl.ds(ebase + j * EC, EC)],
                                  idxbufs[b], sems_i[b]).wait()
            pltpu.make_async_copy(attrt_h.at[pl.ds(vbase + j * EC, EC)],
                                  valbufs[b], sems_v[b]).wait()

        start1(0, 0)

        def body1(j2, carry):
            for b in (0, 1):
                j = j2 * 2 + b

                @pl.when(j + 1 < NJ1)
                def _():
                    start1(j + 1, 1 - b)

                wait1(j, b)

                @plsc.parallel_loop(0, EC // 16, 1, unroll=8)
                def _(kk):
                    o = kk * 16
                    iv = idxbufs[b][pl.ds(o, 16)]
                    vv = valbufs[b][pl.ds(o, 16)]
                    plsc.addupdate_scatter(hist_v, [iv], vv)
            return carry

        lax.fori_loop(0, NJ1 // 2, body1, 0)
        pltpu.sync_copy(hist_v, sums_h.at[w])

        pltpu.sync_copy(z_h, hist_v)
        sbase = w * EDGES_PER_TILE
        ones16 = jnp.full((16,), 1.0, dtype=jnp.float32)

        def start2(j, b):
            pltpu.async_copy(col_h.at[pl.ds(sbase + j * EC, EC)],
                             idxbufs[b], sems_i[b])

        def wait2(j, b):
            pltpu.make_async_copy(col_h.at[pl.ds(sbase + j * EC, EC)],
                                  idxbufs[b], sems_i[b]).wait()

        start2(0, 0)

        def body2(j2, carry):
            for b in (0, 1):
                j = j2 * 2 + b

                @pl.when(j + 1 < NJ2)
                def _():
                    start2(j + 1, 1 - b)

                wait2(j, b)

                @plsc.parallel_loop(0, EC // 16, 1, unroll=8)
                def _(kk):
                    o = kk * 16
                    iv = idxbufs[b][pl.ds(o, 16)]
                    plsc.addupdate_scatter(hist_v, [iv], ones16)
            return carry

        lax.fori_loop(0, NJ2 // 2, body2, 0)
        pltpu.sync_copy(hist_v, cnts_h.at[w])

    return k(col, attr_t, zeros_flat)


def _tc_combine(x, s0, s1, cnts, wx_t, wa_t, b2):
    def body(x_r, s0_r, s1_r, c_r, wx_r, wa_r, b_r, o_r):
        ones_col = jnp.ones((NW, 1), dtype=jnp.float32)
        cnt = jnp.dot(c_r[...], ones_col, preferred_element_type=jnp.float32)
        inv = 1.0 / jnp.maximum(cnt, 1.0)
        agg = (s0_r[...] + s1_r[...]) * inv
        acc = jnp.dot(x_r[...], wx_r[...], preferred_element_type=jnp.float32)
        acc = acc + jnp.dot(agg, wa_r[...], preferred_element_type=jnp.float32)
        o_r[...] = acc + b_r[...]

    grid = N_NODES // BLK
    return pl.pallas_call(
        body,
        grid=(grid,),
        in_specs=[
            pl.BlockSpec((BLK, D_FEAT), lambda i: (i, 0)),
            pl.BlockSpec((BLK, D_EDGE), lambda i: (i, 0)),
            pl.BlockSpec((BLK, D_EDGE), lambda i: (i, 0)),
            pl.BlockSpec((BLK, NW), lambda i: (i, 0)),
            pl.BlockSpec((D_FEAT, D_FEAT), lambda i: (0, 0)),
            pl.BlockSpec((D_EDGE, D_FEAT), lambda i: (0, 0)),
            pl.BlockSpec((1, D_FEAT), lambda i: (0, 0)),
        ],
        out_specs=pl.BlockSpec((BLK, D_FEAT), lambda i: (i, 0)),
        out_shape=jax.ShapeDtypeStruct((N_NODES, D_FEAT), jnp.float32),
    )(x, s0, s1, cnts, wx_t, wa_t, b2)


@jax.jit
def kernel(x, edge_index, edge_attr, W, b):
    col = edge_index[1]
    attr_t = edge_attr.T.reshape(-1)
    zeros_flat = jnp.zeros((N_PAD,), jnp.float32)
    sums_f, cnts_f = _sc_segment_sum(col, attr_t, zeros_flat)
    s0 = sums_f[:NS, :N_NODES].T
    s1 = sums_f[NS:, :N_NODES].T
    cnts = cnts_f[:, :N_NODES].T
    wt = W.T
    return _tc_combine(x, s0, s1, cnts, wt[:D_FEAT], wt[D_FEAT:], b[None, :])
